# Initial kernel scaffold; baseline (speedup 1.0000x reference)
import jax
import jax.numpy as jnp
from jax.experimental import pallas as pl


def kernel(x, Wr, W1, W2, W3):
    raise NotImplementedError("write your pallas kernel here")


# speedup vs baseline: 1.4506x; 1.4506x over previous
import functools

import jax
import jax.numpy as jnp
from jax.experimental import pallas as pl
from jax.experimental.pallas import tpu as pltpu

D_MODEL = 1024
D_FF = 4096
N_EXPERTS = 8
TOP_K = 2
LB_COEF = 0.01

F_BLK = 1024
N_F = D_FF // F_BLK


def _router_body(x_ref, wr_ref, w_ref, loss_ref):
    x = x_ref[...]
    wr = wr_ref[...]
    logits = jax.lax.dot_general(x, wr, (((1,), (1,)), ((), ())),
                                 preferred_element_type=jnp.float32)
    eids = jax.lax.broadcasted_iota(jnp.int32, logits.shape, 1)
    m1 = jnp.max(logits, axis=-1, keepdims=True)
    is1 = logits == m1
    first1 = jnp.min(jnp.where(is1, eids, N_EXPERTS), axis=-1, keepdims=True)
    oh1 = eids == first1
    neg = jnp.float32(-1e30)
    rem = jnp.where(oh1, neg, logits)
    m2 = jnp.max(rem, axis=-1, keepdims=True)
    first2 = jnp.min(jnp.where(rem == m2, eids, N_EXPERTS), axis=-1, keepdims=True)
    oh2 = eids == first2
    e2 = jnp.exp(m2 - m1)
    denom = 1.0 + e2
    p1 = 1.0 / denom
    p2 = e2 / denom
    w_ref[...] = jnp.where(oh1, p1, 0.0) + jnp.where(oh2, p2, 0.0)
    z = jnp.exp(logits - m1)
    rp = z / jnp.sum(z, axis=-1, keepdims=True)
    ep = jnp.mean(rp, axis=0)
    loss_ref[0, 0] = LB_COEF * N_EXPERTS * jnp.sum(ep * ep)


def _expert_body(w_all_ref, x_ref, w1_ref, w2_ref, w3_ref, out_ref):
    e = pl.program_id(0)
    f = pl.program_id(1)

    @pl.when(jnp.logical_and(e == 0, f == 0))
    def _():
        out_ref[...] = jnp.zeros_like(out_ref)

    x = x_ref[...]
    w1 = w1_ref[0]
    w3 = w3_ref[0]
    w2 = w2_ref[0]
    gate = jax.lax.dot_general(x, w1, (((1,), (1,)), ((), ())),
                               preferred_element_type=jnp.float32)
    up = jax.lax.dot_general(x, w3, (((1,), (1,)), ((), ())),
                             preferred_element_type=jnp.float32)
    eids = jax.lax.broadcasted_iota(jnp.int32, w_all_ref.shape, 1)
    wcol = jnp.sum(jnp.where(eids == e, w_all_ref[...], 0.0), axis=-1,
                   keepdims=True)
    h = (gate * jax.nn.sigmoid(gate)) * up * wcol
    out_ref[...] += jax.lax.dot_general(h, w2, (((1,), (1,)), ((), ())),
                                        preferred_element_type=jnp.float32)


@jax.jit
def kernel(x, Wr, W1, W2, W3):
    B, T, C = x.shape
    x_flat = x.reshape(T, C)

    w_all, loss = pl.pallas_call(
        _router_body,
        out_shape=(
            jax.ShapeDtypeStruct((T, N_EXPERTS), jnp.float32),
            jax.ShapeDtypeStruct((1, 1), jnp.float32),
        ),
        in_specs=[
            pl.BlockSpec((T, C), lambda: (0, 0)),
            pl.BlockSpec((N_EXPERTS, C), lambda: (0, 0)),
        ],
        out_specs=(
            pl.BlockSpec((T, N_EXPERTS), lambda: (0, 0)),
            pl.BlockSpec(memory_space=pltpu.SMEM),
        ),
    )(x_flat, Wr)

    out = pl.pallas_call(
        _expert_body,
        grid=(N_EXPERTS, N_F),
        out_shape=jax.ShapeDtypeStruct((T, C), jnp.float32),
        in_specs=[
            pl.BlockSpec((T, N_EXPERTS), lambda e, f: (0, 0)),
            pl.BlockSpec((T, C), lambda e, f: (0, 0)),
            pl.BlockSpec((1, F_BLK, C), lambda e, f: (e, f, 0)),
            pl.BlockSpec((1, C, F_BLK), lambda e, f: (e, 0, f)),
            pl.BlockSpec((1, F_BLK, C), lambda e, f: (e, f, 0)),
        ],
        out_specs=pl.BlockSpec((T, C), lambda e, f: (0, 0)),
    )(w_all, x_flat, W1, W2, W3)

    return out.reshape(B, T, C), loss[0, 0]
